# baseline (speedup 1.0000x reference)
import functools

import jax
import jax.numpy as jnp
from jax import lax
from jax.experimental import pallas as pl
from jax.experimental.pallas import tpu as pltpu
from jax.experimental.pallas import tpu_sc as plsc

_NUM_CORES = 2
_NUM_SUBCORES = 16
_NW = _NUM_CORES * _NUM_SUBCORES
_CHUNK = 128


@functools.cache
def _make_gather(batch: int, dim: int):
    b_per_w = batch // _NW
    n_chunks = b_per_w // _CHUNK
    mesh = plsc.VectorSubcoreMesh(
        core_axis_name="c", subcore_axis_name="s",
        num_cores=_NUM_CORES, num_subcores=_NUM_SUBCORES)

    @functools.partial(
        pl.kernel,
        out_type=jax.ShapeDtypeStruct((batch, dim), jnp.float32),
        mesh=mesh,
        scratch_types=[
            pltpu.VMEM((n_chunks, _CHUNK), jnp.int32),
            pltpu.VMEM((b_per_w, dim), jnp.float32),
            pltpu.SemaphoreType.DMA,
        ],
        compiler_params=pltpu.CompilerParams(use_tc_tiling_on_sc=False),
    )
    def grab(ids_hbm, table_hbm, out_hbm, idx_v, rows_v, sem):
        wid = lax.axis_index("s") * _NUM_CORES + lax.axis_index("c")
        base = wid * b_per_w
        pltpu.sync_copy(ids_hbm.at[wid], idx_v)
        copies = [
            pltpu.async_copy(
                table_hbm.at[idx_v.at[j]],
                rows_v.at[pl.ds(j * _CHUNK, _CHUNK)],
                sem)
            for j in range(n_chunks)
        ]
        for c in copies:
            c.wait()
        pltpu.sync_copy(rows_v, out_hbm.at[pl.ds(base, b_per_w)])

    return grab


def kernel(junction_ids, table):
    batch, = junction_ids.shape
    _, dim = table.shape
    ids = junction_ids.astype(jnp.int32).reshape(_NW, batch // (_NW * _CHUNK), _CHUNK)
    return _make_gather(batch, dim)(ids, table)


# speedup vs baseline: 1.2930x; 1.2930x over previous
import functools

import jax
import jax.numpy as jnp
from jax import lax
from jax.experimental import pallas as pl
from jax.experimental.pallas import tpu as pltpu
from jax.experimental.pallas import tpu_sc as plsc

_NUM_CORES = 2
_NUM_SUBCORES = 16
_NW = _NUM_CORES * _NUM_SUBCORES
_LANES = 16


@functools.cache
def _make_gather(batch: int, dim: int):
    b_per_w = batch // _NW
    mesh = plsc.VectorSubcoreMesh(
        core_axis_name="c", subcore_axis_name="s",
        num_cores=_NUM_CORES, num_subcores=_NUM_SUBCORES)

    @functools.partial(
        pl.kernel,
        out_type=jax.ShapeDtypeStruct((batch, dim), jnp.float32),
        mesh=mesh,
        scratch_types=[
            pltpu.VMEM((b_per_w,), jnp.int32),
            pltpu.VMEM((b_per_w, dim), jnp.float32),
            pltpu.SemaphoreType.DMA,
        ],
        compiler_params=pltpu.CompilerParams(needs_layout_passes=False),
    )
    def grab(ids_hbm, table_hbm, out_hbm, ids_v, rows_v, sem):
        wid = lax.axis_index("s") * _NUM_CORES + lax.axis_index("c")
        base = wid * b_per_w
        pltpu.sync_copy(ids_hbm.at[pl.ds(base, b_per_w)], ids_v)
        lanes = lax.iota(jnp.int32, _LANES)

        def body(i, carry):
            vec = ids_v[pl.ds((i // _LANES) * _LANES, _LANES)]
            lane = lax.rem(i, _LANES)
            sid = jnp.sum(jnp.where(lanes == lane, vec, 0))
            g = lax.shift_right_logical(sid, 3)
            r = lax.bitwise_and(sid, 7)
            pltpu.async_copy(table_hbm.at[g, r], rows_v.at[i], sem).wait()
            return carry

        lax.fori_loop(0, b_per_w, body, jnp.int32(0))
        pltpu.sync_copy(rows_v, out_hbm.at[pl.ds(base, b_per_w)])

    return grab


def kernel(junction_ids, table):
    batch, = junction_ids.shape
    nrows, dim = table.shape
    ids = junction_ids.astype(jnp.int32)
    table3 = table.reshape(nrows // 8, 8, dim)
    return _make_gather(batch, dim)(ids, table3)


# speedup vs baseline: 1.7343x; 1.3413x over previous
import functools

import jax
import jax.numpy as jnp
from jax import lax
from jax.experimental import pallas as pl
from jax.experimental.pallas import tpu as pltpu
from jax.experimental.pallas import tpu_sc as plsc

_NUM_CORES = 2
_NUM_SUBCORES = 16
_NW = _NUM_CORES * _NUM_SUBCORES


@functools.cache
def _make_gather(batch: int, dim: int):
    b_per_w = batch // _NW
    mesh = plsc.VectorSubcoreMesh(
        core_axis_name="c", subcore_axis_name="s",
        num_cores=_NUM_CORES, num_subcores=_NUM_SUBCORES)

    @functools.partial(
        pl.kernel,
        out_type=jax.ShapeDtypeStruct((batch, dim), jnp.float32),
        mesh=mesh,
        scratch_types=[
            pltpu.VMEM((b_per_w,), jnp.int32),
            pltpu.VMEM((b_per_w, dim), jnp.float32),
            pltpu.SemaphoreType.DMA,
        ],
        compiler_params=pltpu.CompilerParams(needs_layout_passes=False),
    )
    def grab(ids_hbm, table_hbm, out_hbm, ids_v, rows_v, sem):
        wid = lax.axis_index("s") * _NUM_CORES + lax.axis_index("c")
        base = wid * b_per_w
        pltpu.sync_copy(ids_hbm.at[pl.ds(base, b_per_w)], ids_v)
        lanes = lax.iota(jnp.int32, 16)

        def body(v, carry, lane=None):
            vec = ids_v[pl.ds(v * 16, 16)]
            for lane in range(16):
                sid = jnp.sum(jnp.where(lanes == lane, vec, 0))
                pltpu.async_copy(table_hbm.at[sid], rows_v.at[v * 16 + lane], sem)
            return carry

        lax.fori_loop(0, b_per_w // 16, body, jnp.int32(0))
        pltpu.make_async_copy(
            out_hbm.at[pl.ds(base, b_per_w)], rows_v, sem).wait()
        pltpu.sync_copy(rows_v, out_hbm.at[pl.ds(base, b_per_w)])

    return grab


def kernel(junction_ids, table):
    batch, = junction_ids.shape
    _, dim = table.shape
    ids = junction_ids.astype(jnp.int32)
    return _make_gather(batch, dim)(ids, table)


# speedup vs baseline: 1.7344x; 1.0000x over previous
import functools

import jax
import jax.numpy as jnp
from jax import lax
from jax.experimental import pallas as pl
from jax.experimental.pallas import tpu as pltpu
from jax.experimental.pallas import tpu_sc as plsc

_NUM_CORES = 2
_NUM_SUBCORES = 16
_NW = _NUM_CORES * _NUM_SUBCORES


@functools.cache
def _make_gather(batch: int, dim: int):
    b_per_w = batch // _NW
    mesh = plsc.VectorSubcoreMesh(
        core_axis_name="c", subcore_axis_name="s",
        num_cores=_NUM_CORES, num_subcores=_NUM_SUBCORES)

    @functools.partial(
        pl.kernel,
        out_type=jax.ShapeDtypeStruct((batch, dim), jnp.float32),
        mesh=mesh,
        scratch_types=[
            pltpu.VMEM((b_per_w,), jnp.int32),
            pltpu.VMEM((b_per_w, dim), jnp.float32),
            pltpu.SemaphoreType.DMA,
            pltpu.SemaphoreType.DMA,
            pltpu.SemaphoreType.DMA,
            pltpu.SemaphoreType.DMA,
        ],
        compiler_params=pltpu.CompilerParams(needs_layout_passes=False),
    )
    def grab(ids_hbm, table_hbm, out_hbm, ids_v, rows_v, s0, s1, s2, s3):
        wid = lax.axis_index("s") * _NUM_CORES + lax.axis_index("c")
        base = wid * b_per_w
        sems = (s0, s1, s2, s3)
        pltpu.sync_copy(ids_hbm.at[pl.ds(base, b_per_w)], ids_v)
        lanes = lax.iota(jnp.int32, 16)

        def body(v, carry, lane=None):
            vec = ids_v[pl.ds(v * 16, 16)]
            for lane in range(16):
                sid = jnp.sum(jnp.where(lanes == lane, vec, 0))
                pltpu.async_copy(table_hbm.at[sid], rows_v.at[v * 16 + lane],
                                 sems[lane % 4])
            return carry

        lax.fori_loop(0, b_per_w // 16, body, jnp.int32(0))
        quarter = rows_v.at[pl.ds(0, b_per_w // 4)]
        dummy = out_hbm.at[pl.ds(base, b_per_w // 4)]
        for s in sems:
            pltpu.make_async_copy(dummy, quarter, s).wait()
        pltpu.sync_copy(rows_v, out_hbm.at[pl.ds(base, b_per_w)])

    return grab


def kernel(junction_ids, table):
    batch, = junction_ids.shape
    _, dim = table.shape
    ids = junction_ids.astype(jnp.int32)
    return _make_gather(batch, dim)(ids, table)
